# baseline (speedup 1.0000x reference)
import jax
import jax.numpy as jnp
from jax import lax
from jax.experimental import pallas as pl
from jax.experimental.pallas import tpu as pltpu
from jax.experimental.pallas import tpu_sc as plsc

NUM_CORES = 2
NUM_SUBCORES = 16
NW = NUM_CORES * NUM_SUBCORES
LANES = 16
IDX_CHUNK = 128


def _cf_score_kernel(user_emb, item_emb, user_bias2, item_bias2,
                     user_id, pos_item_id, neg_item_id, out_hbm,
                     idx_u, idx_p, idx_n, idxb_u, idxb_p, idxb_n,
                     u_rows, p_rows, n_rows, ub_rows, pb_rows, nb_rows,
                     po_v, no_v,
                     sem0, sem1, sem2, sem3, sem4, sem5):
    b = user_id.shape[0]
    bw = b // NW
    f = user_emb.shape[1]
    nch = bw // IDX_CHUNK

    wid = lax.axis_index("s") * NUM_CORES + lax.axis_index("c")
    base = wid * bw

    pltpu.sync_copy(user_id.at[pl.ds(base, bw)], idx_u)
    pltpu.sync_copy(pos_item_id.at[pl.ds(base, bw)], idx_p)
    pltpu.sync_copy(neg_item_id.at[pl.ds(base, bw)], idx_n)

    cps = []
    for j in range(nch):
        sl = pl.ds(j * IDX_CHUNK, IDX_CHUNK)
        cps.append(pltpu.async_copy(user_emb.at[idx_u.at[sl]], u_rows.at[sl], sem0))
        cps.append(pltpu.async_copy(item_emb.at[idx_p.at[sl]], p_rows.at[sl], sem1))
        cps.append(pltpu.async_copy(item_emb.at[idx_n.at[sl]], n_rows.at[sl], sem2))

    for j in range(bw // LANES):
        sl = pl.ds(j * LANES, LANES)
        idxb_u[sl] = lax.shift_right_logical(idx_u[sl], 4)
        idxb_p[sl] = lax.shift_right_logical(idx_p[sl], 4)
        idxb_n[sl] = lax.shift_right_logical(idx_n[sl], 4)

    for j in range(nch):
        sl = pl.ds(j * IDX_CHUNK, IDX_CHUNK)
        cps.append(pltpu.async_copy(user_bias2.at[idxb_u.at[sl]], ub_rows.at[sl], sem3))
        cps.append(pltpu.async_copy(item_bias2.at[idxb_p.at[sl]], pb_rows.at[sl], sem4))
        cps.append(pltpu.async_copy(item_bias2.at[idxb_n.at[sl]], nb_rows.at[sl], sem5))

    for cp in cps:
        cp.wait()

    lane_iota = lax.iota(jnp.int32, LANES)
    lane_mask = jnp.full((LANES,), LANES - 1, jnp.int32)

    def body(g, carry):
        row0 = g * LANES
        rows = row0 + lane_iota
        accp = jnp.zeros((LANES,), jnp.float32)
        accn = jnp.zeros((LANES,), jnp.float32)
        for fc in range(f):
            colf = jnp.full((LANES,), fc, jnp.int32)
            uv = plsc.load_gather(u_rows, [rows, colf])
            pv = plsc.load_gather(p_rows, [rows, colf])
            nv = plsc.load_gather(n_rows, [rows, colf])
            accp = accp + uv * pv
            accn = accn + uv * nv
        rel = lane_iota
        ubv = plsc.load_gather(ub_rows, [row0 + rel, idx_u[pl.ds(row0, LANES)] & lane_mask])
        pbv = plsc.load_gather(pb_rows, [row0 + rel, idx_p[pl.ds(row0, LANES)] & lane_mask])
        nbv = plsc.load_gather(nb_rows, [row0 + rel, idx_n[pl.ds(row0, LANES)] & lane_mask])
        po_v[pl.ds(row0, LANES)] = accp + ubv + pbv
        no_v[pl.ds(row0, LANES)] = accn + ubv + nbv
        return carry

    lax.fori_loop(0, bw // LANES, body, 0)

    pltpu.sync_copy(po_v, out_hbm.at[0, pl.ds(base, bw)])
    pltpu.sync_copy(no_v, out_hbm.at[1, pl.ds(base, bw)])


@jax.jit
def kernel(user_emb, item_emb, user_bias, item_bias, user_id, pos_item_id, neg_item_id):
    b = user_id.shape[0]
    bw = b // NW
    f = user_emb.shape[1]
    user_bias2 = user_bias.reshape(-1, LANES)
    item_bias2 = item_bias.reshape(-1, LANES)
    mesh = plsc.VectorSubcoreMesh(
        core_axis_name="c", subcore_axis_name="s",
        num_cores=NUM_CORES, num_subcores=NUM_SUBCORES)
    run = pl.kernel(
        _cf_score_kernel,
        out_type=jax.ShapeDtypeStruct((2, b), jnp.float32),
        mesh=mesh,
        compiler_params=pltpu.CompilerParams(
            needs_layout_passes=False, use_tc_tiling_on_sc=False),
        scratch_types=[
            pltpu.VMEM((bw,), jnp.int32),
            pltpu.VMEM((bw,), jnp.int32),
            pltpu.VMEM((bw,), jnp.int32),
            pltpu.VMEM((bw,), jnp.int32),
            pltpu.VMEM((bw,), jnp.int32),
            pltpu.VMEM((bw,), jnp.int32),
            pltpu.VMEM((bw, f), jnp.float32),
            pltpu.VMEM((bw, f), jnp.float32),
            pltpu.VMEM((bw, f), jnp.float32),
            pltpu.VMEM((bw, LANES), jnp.float32),
            pltpu.VMEM((bw, LANES), jnp.float32),
            pltpu.VMEM((bw, LANES), jnp.float32),
            pltpu.VMEM((bw,), jnp.float32),
            pltpu.VMEM((bw,), jnp.float32),
            pltpu.SemaphoreType.DMA,
            pltpu.SemaphoreType.DMA,
            pltpu.SemaphoreType.DMA,
            pltpu.SemaphoreType.DMA,
            pltpu.SemaphoreType.DMA,
            pltpu.SemaphoreType.DMA,
        ],
    )
    return run(user_emb, item_emb, user_bias2, item_bias2,
               user_id, pos_item_id, neg_item_id)


# speedup vs baseline: 1.2323x; 1.2323x over previous
import jax
import jax.numpy as jnp
from jax import lax
from jax.experimental import pallas as pl
from jax.experimental.pallas import tpu as pltpu
from jax.experimental.pallas import tpu_sc as plsc

NUM_CORES = 2
NUM_SUBCORES = 16
NW = NUM_CORES * NUM_SUBCORES
LANES = 16
IDX_CHUNK = 128


def _cf_score_kernel(user_emb, item_emb, user_bias2, item_bias2,
                     user_id, pos_item_id, neg_item_id, out_hbm,
                     idx_u, idx_p, idx_n, idxb_u, idxb_p, idxb_n,
                     u_rows, p_rows, n_rows, ub_rows, pb_rows, nb_rows,
                     po_v, no_v,
                     sem0, sem1, sem2, sem3, sem4, sem5):
    b = user_id.shape[0]
    bw = b // NW
    f = user_emb.shape[1]
    nch = bw // IDX_CHUNK

    wid = lax.axis_index("s") * NUM_CORES + lax.axis_index("c")
    base = wid * bw

    pltpu.sync_copy(user_id.at[pl.ds(base, bw)], idx_u)
    pltpu.sync_copy(pos_item_id.at[pl.ds(base, bw)], idx_p)
    pltpu.sync_copy(neg_item_id.at[pl.ds(base, bw)], idx_n)

    cps = []
    for j in range(nch):
        sl = pl.ds(j * IDX_CHUNK, IDX_CHUNK)
        cps.append(pltpu.async_copy(user_emb.at[idx_u.at[sl]], u_rows.at[sl], sem0))
        cps.append(pltpu.async_copy(item_emb.at[idx_p.at[sl]], p_rows.at[sl], sem1))
        cps.append(pltpu.async_copy(item_emb.at[idx_n.at[sl]], n_rows.at[sl], sem2))

    for j in range(bw // LANES):
        sl = pl.ds(j * LANES, LANES)
        idxb_u[sl] = lax.shift_right_logical(idx_u[sl], 4)
        idxb_p[sl] = lax.shift_right_logical(idx_p[sl], 4)
        idxb_n[sl] = lax.shift_right_logical(idx_n[sl], 4)

    for j in range(nch):
        sl = pl.ds(j * IDX_CHUNK, IDX_CHUNK)
        cps.append(pltpu.async_copy(user_bias2.at[idxb_u.at[sl]], ub_rows.at[sl], sem3))
        cps.append(pltpu.async_copy(item_bias2.at[idxb_p.at[sl]], pb_rows.at[sl], sem4))
        cps.append(pltpu.async_copy(item_bias2.at[idxb_n.at[sl]], nb_rows.at[sl], sem5))

    for cp in cps:
        cp.wait()

    lane_iota = lax.iota(jnp.int32, LANES)
    lane_mask = jnp.full((LANES,), LANES - 1, jnp.int32)
    feat_mask = jnp.full((LANES,), f - 1, jnp.int32)

    def body(g, carry):
        row0 = g * LANES
        rows = row0 + lane_iota
        accp = jnp.zeros((LANES,), jnp.float32)
        accn = jnp.zeros((LANES,), jnp.float32)
        for fc in range(f):
            colf = (lane_iota + fc) & feat_mask
            uv = plsc.load_gather(u_rows, [rows, colf])
            pv = plsc.load_gather(p_rows, [rows, colf])
            nv = plsc.load_gather(n_rows, [rows, colf])
            accp = accp + uv * pv
            accn = accn + uv * nv
        rel = lane_iota
        ubv = plsc.load_gather(ub_rows, [row0 + rel, idx_u[pl.ds(row0, LANES)] & lane_mask])
        pbv = plsc.load_gather(pb_rows, [row0 + rel, idx_p[pl.ds(row0, LANES)] & lane_mask])
        nbv = plsc.load_gather(nb_rows, [row0 + rel, idx_n[pl.ds(row0, LANES)] & lane_mask])
        po_v[pl.ds(row0, LANES)] = accp + ubv + pbv
        no_v[pl.ds(row0, LANES)] = accn + ubv + nbv
        return carry

    lax.fori_loop(0, bw // LANES, body, 0)

    pltpu.sync_copy(po_v, out_hbm.at[0, pl.ds(base, bw)])
    pltpu.sync_copy(no_v, out_hbm.at[1, pl.ds(base, bw)])


@jax.jit
def kernel(user_emb, item_emb, user_bias, item_bias, user_id, pos_item_id, neg_item_id):
    b = user_id.shape[0]
    bw = b // NW
    f = user_emb.shape[1]
    user_bias2 = user_bias.reshape(-1, LANES)
    item_bias2 = item_bias.reshape(-1, LANES)
    mesh = plsc.VectorSubcoreMesh(
        core_axis_name="c", subcore_axis_name="s",
        num_cores=NUM_CORES, num_subcores=NUM_SUBCORES)
    run = pl.kernel(
        _cf_score_kernel,
        out_type=jax.ShapeDtypeStruct((2, b), jnp.float32),
        mesh=mesh,
        compiler_params=pltpu.CompilerParams(
            needs_layout_passes=False, use_tc_tiling_on_sc=False),
        scratch_types=[
            pltpu.VMEM((bw,), jnp.int32),
            pltpu.VMEM((bw,), jnp.int32),
            pltpu.VMEM((bw,), jnp.int32),
            pltpu.VMEM((bw,), jnp.int32),
            pltpu.VMEM((bw,), jnp.int32),
            pltpu.VMEM((bw,), jnp.int32),
            pltpu.VMEM((bw, f), jnp.float32),
            pltpu.VMEM((bw, f), jnp.float32),
            pltpu.VMEM((bw, f), jnp.float32),
            pltpu.VMEM((bw, LANES), jnp.float32),
            pltpu.VMEM((bw, LANES), jnp.float32),
            pltpu.VMEM((bw, LANES), jnp.float32),
            pltpu.VMEM((bw,), jnp.float32),
            pltpu.VMEM((bw,), jnp.float32),
            pltpu.SemaphoreType.DMA,
            pltpu.SemaphoreType.DMA,
            pltpu.SemaphoreType.DMA,
            pltpu.SemaphoreType.DMA,
            pltpu.SemaphoreType.DMA,
            pltpu.SemaphoreType.DMA,
        ],
    )
    return run(user_emb, item_emb, user_bias2, item_bias2,
               user_id, pos_item_id, neg_item_id)
